# baseline (speedup 1.0000x reference)
import functools

import jax
import jax.numpy as jnp
import numpy as np
from jax import lax
from jax.experimental import pallas as pl
from jax.experimental.pallas import tpu as pltpu
from jax.experimental.pallas import tpu_sc as plsc

_V = 1000000
_D = 128
_B = 4096
_L = 50

_NC = 2
_NS = 16
_NW = _NC * _NS
_BPW = _B // _NW
_IPW = _BPW * _L
_IPC = 128
_CHUNKS = _IPW // _IPC


def _pool_body(idx_hbm, rid_hbm, zero_hbm, table_hbm, pooled_hbm,
               idx_v, rid_v, buf0, buf1, pooled_v, shared_acc,
               sem_g0, sem_g1):
    cid = lax.axis_index("c")
    sid = lax.axis_index("s")
    wid = sid * _NC + cid
    pltpu.sync_copy(idx_hbm.at[wid], idx_v)
    pltpu.sync_copy(rid_hbm.at[sid], rid_v)
    pltpu.sync_copy(zero_hbm, pooled_v)
    pltpu.sync_copy(pooled_v, shared_acc.at[pl.ds(sid * _BPW, _BPW)])

    bufs = (buf0, buf1)
    gsems = (sem_g0, sem_g1)

    def _gather(c, p):
        return pltpu.async_copy(table_hbm.at[idx_v.at[c]], bufs[p], gsems[p])

    _gather(0, 0)
    _gather(1, 1)

    def pair_body(c2, carry):
        for p in range(2):
            c = c2 * 2 + p
            pltpu.make_async_copy(
                table_hbm.at[idx_v.at[c]], bufs[p], gsems[p]
            ).wait()
            pltpu.sync_copy(bufs[p], shared_acc.at[rid_v.at[c]], add=True)

            @pl.when(c + 2 < _CHUNKS)
            def _():
                _gather(c + 2, p)

        return carry

    lax.fori_loop(0, _CHUNKS // 2, pair_body, 0)
    pltpu.sync_copy(shared_acc.at[pl.ds(sid * _BPW, _BPW)], pooled_v)
    pltpu.sync_copy(pooled_v, pooled_hbm.at[pl.ds(wid * _BPW, _BPW)])


_pool = functools.partial(
    pl.kernel,
    mesh=plsc.VectorSubcoreMesh(core_axis_name="c", subcore_axis_name="s"),
    out_type=jax.ShapeDtypeStruct((_B, _D), jnp.float32),
    scratch_types=[
        pltpu.VMEM((_CHUNKS, _IPC), jnp.int32),
        pltpu.VMEM((_CHUNKS, _IPC), jnp.int32),
        pltpu.VMEM((_IPC, _D), jnp.float32),
        pltpu.VMEM((_IPC, _D), jnp.float32),
        pltpu.VMEM((_BPW, _D), jnp.float32),
        pltpu.VMEM_SHARED((_NS * _BPW, _D), jnp.float32),
        pltpu.SemaphoreType.DMA,
        pltpu.SemaphoreType.DMA,
    ],
)(_pool_body)

_RID = (
    (np.arange(_IPW, dtype=np.int32) // _L).reshape(_CHUNKS, _IPC)[None, :, :]
    + (np.arange(_NS, dtype=np.int32) * _BPW)[:, None, None]
)


def _linear_body(pooled_ref, wt_ref, bias_ref, out_ref):
    out_ref[...] = (
        jnp.dot(pooled_ref[...], wt_ref[...], preferred_element_type=jnp.float32)
        + bias_ref[...]
    )


def kernel(s, table, W, b):
    s32 = s.astype(jnp.int32)
    flat_idx = s32.reshape(_NW, _CHUNKS, _IPC)
    rid = jnp.asarray(_RID)
    zeros = jnp.zeros((_BPW, _D), jnp.float32)
    pooled = _pool(flat_idx, rid, zeros, table)
    out = pl.pallas_call(
        _linear_body,
        out_shape=jax.ShapeDtypeStruct((_B, 2), jnp.float32),
    )(pooled, W.T.astype(jnp.float32), (_L * b).reshape(1, 2).astype(jnp.float32))
    return out
